# baseline (speedup 1.0000x reference)
import functools

import jax
import jax.numpy as jnp
from jax import lax
from jax.experimental import pallas as pl
from jax.experimental.pallas import tpu as pltpu
from jax.experimental.pallas import tpu_sc as plsc

EMBED_DIM = 32
D_MODEL = 64
LANES = 128
PACK = LANES // EMBED_DIM

NC = 2
NS = 16
NW = NC * NS

CHUNK = 2048
IDX_W = 128
K = CHUNK // IDX_W

VBLK = 4096
TBLK = 16384


def _tr_body(tt_ref, out_ref):
    jj = lax.broadcasted_iota(jnp.int32, (EMBED_DIM, LANES), 1)
    ii = lax.broadcasted_iota(jnp.int32, (EMBED_DIM, LANES), 0)
    eyes = [(ii == jj - u * EMBED_DIM).astype(jnp.float32) for u in range(PACK)]
    for h in range(TBLK // VBLK):
        acc = None
        for u in range(PACK):
            lo = h * VBLK + u * (VBLK // PACK)
            s = lax.dot_general(
                tt_ref[:, lo:lo + VBLK // PACK],
                eyes[u], (((0,), (0,)), ((), ())),
                preferred_element_type=jnp.float32)
            acc = s if acc is None else acc + s
        out_ref[h * (VBLK // PACK):(h + 1) * (VBLK // PACK), :] = (
            acc.astype(jnp.bfloat16))


def _tc_table_linear(tableT, v_pad):
    grid = (v_pad // TBLK,)
    return pl.pallas_call(
        _tr_body,
        grid=grid,
        in_specs=[pl.BlockSpec((EMBED_DIM, TBLK), lambda i: (0, i))],
        out_specs=pl.BlockSpec((TBLK // PACK, LANES), lambda i: (i, 0)),
        out_shape=jax.ShapeDtypeStruct((v_pad // PACK, LANES), jnp.bfloat16),
    )(tableT)


def _gather_body(table_hbm, idx_hbm, out_hbm, idx_m, idx_j, rows_v, sem, *,
                 n_chunks_total):
    wid = lax.axis_index("s") * NC + lax.axis_index("c")
    c_lo = wid * n_chunks_total // NW
    c_hi = (wid + 1) * n_chunks_total // NW
    l16 = lax.iota(jnp.int32, 16)
    lmod = l16 & 3
    ivecs = [q * 4 + (l16 >> 2) for q in range(4)]

    def chunk_step(c, carry):
        base = c * CHUNK
        pltpu.sync_copy(idx_hbm.at[pl.ds(pl.multiple_of(base, 8), CHUNK)],
                        idx_m)

        def perm_step(g, carry2):
            a = [idx_m[pl.ds(u * (CHUNK // 4) + g * 16, 16)]
                 for u in range(4)]
            for q in range(4):
                t = [au[ivecs[q]] for au in a]
                out_q = jnp.where(lmod == 0, t[0],
                                  jnp.where(lmod == 1, t[1],
                                            jnp.where(lmod == 2, t[2],
                                                      t[3])))
                idx_j[pl.ds(g * 64 + q * 16, 16)] = out_q
            return carry2

        lax.fori_loop(0, CHUNK // 64, perm_step, 0)
        copies = [
            pltpu.async_copy(
                table_hbm.at[idx_j.at[pl.ds(t * IDX_W, IDX_W)]],
                rows_v.at[t],
                sem,
            )
            for t in range(K)
        ]
        for cp in copies:
            cp.wait()
        pltpu.sync_copy(
            rows_v,
            out_hbm.at[pl.ds(pl.multiple_of(base // IDX_W, 8), K)])
        return carry

    lax.fori_loop(c_lo, c_hi, chunk_step, 0)


def _sc_gather(table_rows, idx_flat, n_rows):
    mesh = plsc.VectorSubcoreMesh(core_axis_name="c", subcore_axis_name="s")
    kern = pl.kernel(
        functools.partial(_gather_body, n_chunks_total=n_rows // CHUNK),
        out_type=jax.ShapeDtypeStruct((n_rows // IDX_W, IDX_W, EMBED_DIM),
                                      jnp.bfloat16),
        mesh=mesh,
        scratch_types=[
            pltpu.VMEM((CHUNK,), jnp.int32),
            pltpu.VMEM((CHUNK,), jnp.int32),
            pltpu.VMEM((K, IDX_W, EMBED_DIM), jnp.bfloat16),
            pltpu.SemaphoreType.DMA,
        ],
        compiler_params=pltpu.CompilerParams(use_tc_tiling_on_sc=False),
    )
    return kern(table_rows, idx_flat)


def _mm_body(emb_ref, w_ref, b_ref, *rest):
    out_ref = rest[-1]
    halves = []
    for h in range(2):
        strips = []
        for u in range(PACK):
            p_u = emb_ref[h * (CHUNK // PACK):(h + 1) * (CHUNK // PACK),
                          u * EMBED_DIM:(u + 1) * EMBED_DIM]
            strips.append(
                lax.dot_general(w_ref[...], p_u, (((1,), (1,)), ((), ())),
                                preferred_element_type=jnp.float32))
        halves.extend(strips)
    acc = jnp.concatenate(halves, axis=1)
    out_ref[...] = jnp.maximum(acc + b_ref[:, :1], 0.0)[None]


def _tc_linear_relu_t(emb_packed, Wt, bcol, seq, batch, s_off, total_seq,
                      prev=None):
    grid = (seq,)
    in_specs = [
        pl.BlockSpec((batch // PACK, LANES), lambda s: (s, 0)),
        pl.BlockSpec((D_MODEL, EMBED_DIM), lambda s: (0, 0)),
        pl.BlockSpec((D_MODEL, LANES), lambda s: (0, 0)),
    ]
    args = [emb_packed, Wt, bcol]
    aliases = {}
    if prev is not None:
        in_specs.append(pl.BlockSpec(memory_space=pl.ANY))
        args.append(prev)
        aliases = {3: 0}
    return pl.pallas_call(
        _mm_body,
        grid=grid,
        in_specs=in_specs,
        out_specs=pl.BlockSpec((1, D_MODEL, batch),
                               lambda s: (s + s_off, 0, 0)),
        out_shape=jax.ShapeDtypeStruct((total_seq, D_MODEL, batch),
                                       jnp.float32),
        input_output_aliases=aliases,
    )(*args)


def kernel(x, table, W, b):
    batch, seq = x.shape
    n_rows = batch * seq
    vocab = table.shape[0]
    v_pad = ((vocab + TBLK - 1) // TBLK) * TBLK

    table_lin = _tc_table_linear(table.T, v_pad)
    table_rows = table_lin.reshape(v_pad, EMBED_DIM)
    Wt_b16 = W.T.astype(jnp.bfloat16)

    v = x.T.astype(jnp.int32).reshape(n_rows)
    addr = (v & -VBLK) | ((v & (VBLK // PACK - 1)) << 2) | ((v & (VBLK - 1))
                                                           >> 10)
    n_split = 4
    seq_q = seq // n_split
    rows_q = n_rows // n_split
    Wt = Wt_b16
    bcol = jnp.tile(b.reshape(D_MODEL, 1), (1, LANES))
    out_t = None
    for i in range(n_split):
        addr_i = lax.slice(addr, (i * rows_q,), ((i + 1) * rows_q,))
        emb3 = _sc_gather(table_rows, addr_i, rows_q)
        emb_packed = emb3.reshape(rows_q * EMBED_DIM // LANES, LANES)
        out_t = _tc_linear_relu_t(emb_packed, Wt, bcol, seq_q, batch,
                                  i * seq_q, seq, prev=out_t)
    return lax.transpose(out_t, (2, 0, 1))


# speedup vs baseline: 1.9486x; 1.9486x over previous
import functools

import jax
import jax.numpy as jnp
from jax import lax
from jax.experimental import pallas as pl
from jax.experimental.pallas import tpu as pltpu
from jax.experimental.pallas import tpu_sc as plsc

EMBED_DIM = 32
D_MODEL = 64
LANES = 128
PACK = LANES // EMBED_DIM

NC = 2
NS = 16
NW = NC * NS

CHUNK = 2048
IDX_W = 128
K = CHUNK // IDX_W

VBLK = 4096
TBLK = 16384


def _tr_body(tt_ref, out_ref):
    jj = lax.broadcasted_iota(jnp.int32, (EMBED_DIM, LANES), 1)
    ii = lax.broadcasted_iota(jnp.int32, (EMBED_DIM, LANES), 0)
    eyes = [(ii == jj - u * EMBED_DIM).astype(jnp.float32) for u in range(PACK)]
    for h in range(TBLK // VBLK):
        acc = None
        for u in range(PACK):
            lo = h * VBLK + u * (VBLK // PACK)
            s = lax.dot_general(
                tt_ref[:, lo:lo + VBLK // PACK],
                eyes[u], (((0,), (0,)), ((), ())),
                preferred_element_type=jnp.float32)
            acc = s if acc is None else acc + s
        out_ref[h * (VBLK // PACK):(h + 1) * (VBLK // PACK), :] = acc


def _tc_table_linear(tableT, v_pad):
    grid = (v_pad // TBLK,)
    return pl.pallas_call(
        _tr_body,
        grid=grid,
        in_specs=[pl.BlockSpec((EMBED_DIM, TBLK), lambda i: (0, i))],
        out_specs=pl.BlockSpec((TBLK // PACK, LANES), lambda i: (i, 0)),
        out_shape=jax.ShapeDtypeStruct((v_pad // PACK, LANES), jnp.float32),
    )(tableT)


def _gather_body(table_hbm, idx_hbm, out_hbm, idx_m, idx_j, rows_v, sem, *,
                 n_chunks_total):
    wid = lax.axis_index("s") * NC + lax.axis_index("c")
    c_lo = wid * n_chunks_total // NW
    c_hi = (wid + 1) * n_chunks_total // NW
    l16 = lax.iota(jnp.int32, 16)
    lmod = l16 & 3
    ivecs = [q * 4 + (l16 >> 2) for q in range(4)]

    def chunk_step(c, carry):
        base = c * CHUNK
        pltpu.sync_copy(idx_hbm.at[pl.ds(pl.multiple_of(base, 8), CHUNK)],
                        idx_m)

        def perm_step(g, carry2):
            a = [idx_m[pl.ds(u * (CHUNK // 4) + g * 16, 16)]
                 for u in range(4)]
            for q in range(4):
                t = [au[ivecs[q]] for au in a]
                out_q = jnp.where(lmod == 0, t[0],
                                  jnp.where(lmod == 1, t[1],
                                            jnp.where(lmod == 2, t[2],
                                                      t[3])))
                idx_j[pl.ds(g * 64 + q * 16, 16)] = out_q
            return carry2

        lax.fori_loop(0, CHUNK // 64, perm_step, 0)
        copies = [
            pltpu.async_copy(
                table_hbm.at[idx_j.at[pl.ds(t * IDX_W, IDX_W)]],
                rows_v.at[t],
                sem,
            )
            for t in range(K)
        ]
        for cp in copies:
            cp.wait()
        pltpu.sync_copy(
            rows_v,
            out_hbm.at[pl.ds(pl.multiple_of(base // IDX_W, 8), K)])
        return carry

    lax.fori_loop(c_lo, c_hi, chunk_step, 0)


def _sc_gather(table_rows, idx_flat, n_rows):
    mesh = plsc.VectorSubcoreMesh(core_axis_name="c", subcore_axis_name="s")
    kern = pl.kernel(
        functools.partial(_gather_body, n_chunks_total=n_rows // CHUNK),
        out_type=jax.ShapeDtypeStruct((n_rows // IDX_W, IDX_W, EMBED_DIM),
                                      jnp.float32),
        mesh=mesh,
        scratch_types=[
            pltpu.VMEM((CHUNK,), jnp.int32),
            pltpu.VMEM((CHUNK,), jnp.int32),
            pltpu.VMEM((K, IDX_W, EMBED_DIM), jnp.float32),
            pltpu.SemaphoreType.DMA,
        ],
        compiler_params=pltpu.CompilerParams(use_tc_tiling_on_sc=False),
    )
    return kern(table_rows, idx_flat)


def _mm_body(emb_ref, w_ref, b_ref, *rest):
    out_ref = rest[-1]
    halves = []
    for h in range(2):
        strips = []
        for u in range(PACK):
            p_u = emb_ref[h * (CHUNK // PACK):(h + 1) * (CHUNK // PACK),
                          u * EMBED_DIM:(u + 1) * EMBED_DIM]
            strips.append(
                lax.dot_general(w_ref[...], p_u, (((1,), (1,)), ((), ())),
                                preferred_element_type=jnp.float32))
        halves.extend(strips)
    acc = jnp.concatenate(halves, axis=1)
    out_ref[...] = jnp.maximum(acc + b_ref[:, :1], 0.0)[None]


def _tc_linear_relu_t(emb_packed, Wt, bcol, seq, batch, s_off, total_seq,
                      prev=None):
    grid = (seq,)
    in_specs = [
        pl.BlockSpec((batch // PACK, LANES), lambda s: (s, 0)),
        pl.BlockSpec((D_MODEL, EMBED_DIM), lambda s: (0, 0)),
        pl.BlockSpec((D_MODEL, LANES), lambda s: (0, 0)),
    ]
    args = [emb_packed, Wt, bcol]
    aliases = {}
    if prev is not None:
        in_specs.append(pl.BlockSpec(memory_space=pl.ANY))
        args.append(prev)
        aliases = {3: 0}
    return pl.pallas_call(
        _mm_body,
        grid=grid,
        in_specs=in_specs,
        out_specs=pl.BlockSpec((1, D_MODEL, batch),
                               lambda s: (s + s_off, 0, 0)),
        out_shape=jax.ShapeDtypeStruct((total_seq, D_MODEL, batch),
                                       jnp.float32),
        input_output_aliases=aliases,
    )(*args)


def kernel(x, table, W, b):
    batch, seq = x.shape
    n_rows = batch * seq
    vocab = table.shape[0]
    v_pad = ((vocab + TBLK - 1) // TBLK) * TBLK

    table_lin = _tc_table_linear(table.T, v_pad)
    table_rows = table_lin.reshape(v_pad, EMBED_DIM)

    v = x.T.astype(jnp.int32).reshape(n_rows)
    addr = (v & -VBLK) | ((v & (VBLK // PACK - 1)) << 2) | ((v & (VBLK - 1))
                                                           >> 10)
    n_split = 4
    seq_q = seq // n_split
    rows_q = n_rows // n_split
    Wt = W.T
    bcol = jnp.tile(b.reshape(D_MODEL, 1), (1, LANES))
    out_t = None
    for i in range(n_split):
        addr_i = lax.slice(addr, (i * rows_q,), ((i + 1) * rows_q,))
        emb3 = _sc_gather(table_rows, addr_i, rows_q)
        emb_packed = emb3.reshape(rows_q * EMBED_DIM // LANES, LANES)
        out_t = _tc_linear_relu_t(emb_packed, Wt, bcol, seq_q, batch,
                                  i * seq_q, seq, prev=out_t)
    return lax.transpose(out_t, (2, 0, 1))


# speedup vs baseline: 2.0158x; 1.0345x over previous
import functools

import jax
import jax.numpy as jnp
from jax import lax
from jax.experimental import pallas as pl
from jax.experimental.pallas import tpu as pltpu
from jax.experimental.pallas import tpu_sc as plsc

EMBED_DIM = 32
D_MODEL = 64
LANES = 128
PACK = LANES // EMBED_DIM

NC = 2
NS = 16
NW = NC * NS

CHUNK = 2048
IDX_W = 128
K = CHUNK // IDX_W

VBLK = 4096
TBLK = 32768


def _tr_body(tt_ref, out_ref):
    jj = lax.broadcasted_iota(jnp.int32, (EMBED_DIM, LANES), 1)
    ii = lax.broadcasted_iota(jnp.int32, (EMBED_DIM, LANES), 0)
    eyes = [(ii == jj - u * EMBED_DIM).astype(jnp.float32) for u in range(PACK)]
    for h in range(TBLK // VBLK):
        acc = None
        for u in range(PACK):
            lo = h * VBLK + u * (VBLK // PACK)
            s = lax.dot_general(
                tt_ref[:, lo:lo + VBLK // PACK],
                eyes[u], (((0,), (0,)), ((), ())),
                preferred_element_type=jnp.float32)
            acc = s if acc is None else acc + s
        out_ref[h * (VBLK // PACK):(h + 1) * (VBLK // PACK), :] = acc


def _tc_table_linear(tableT, v_pad):
    grid = (v_pad // TBLK,)
    return pl.pallas_call(
        _tr_body,
        grid=grid,
        in_specs=[pl.BlockSpec((EMBED_DIM, TBLK), lambda i: (0, i))],
        out_specs=pl.BlockSpec((TBLK // PACK, LANES), lambda i: (i, 0)),
        out_shape=jax.ShapeDtypeStruct((v_pad // PACK, LANES), jnp.float32),
    )(tableT)


def _gather_body(table_hbm, idx_hbm, out_hbm, idx_m, idx_j, rows_v, sem, *,
                 n_chunks_total):
    wid = lax.axis_index("s") * NC + lax.axis_index("c")
    c_lo = wid * n_chunks_total // NW
    c_hi = (wid + 1) * n_chunks_total // NW
    l16 = lax.iota(jnp.int32, 16)
    lmod = l16 & 3
    ivecs = [q * 4 + (l16 >> 2) for q in range(4)]

    def chunk_step(c, carry):
        base = c * CHUNK
        pltpu.sync_copy(idx_hbm.at[pl.ds(pl.multiple_of(base, 8), CHUNK)],
                        idx_m)

        def perm_step(g, carry2):
            a = [idx_m[pl.ds(u * (CHUNK // 4) + g * 16, 16)]
                 for u in range(4)]
            for q in range(4):
                t = [au[ivecs[q]] for au in a]
                out_q = jnp.where(lmod == 0, t[0],
                                  jnp.where(lmod == 1, t[1],
                                            jnp.where(lmod == 2, t[2],
                                                      t[3])))
                idx_j[pl.ds(g * 64 + q * 16, 16)] = out_q
            return carry2

        lax.fori_loop(0, CHUNK // 64, perm_step, 0)
        copies = [
            pltpu.async_copy(
                table_hbm.at[idx_j.at[pl.ds(t * IDX_W, IDX_W)]],
                rows_v.at[t],
                sem,
            )
            for t in range(K)
        ]
        for cp in copies:
            cp.wait()
        pltpu.sync_copy(
            rows_v,
            out_hbm.at[pl.ds(pl.multiple_of(base // IDX_W, 8), K)])
        return carry

    lax.fori_loop(c_lo, c_hi, chunk_step, 0)


def _sc_gather(table_rows, idx_flat, n_rows):
    mesh = plsc.VectorSubcoreMesh(core_axis_name="c", subcore_axis_name="s")
    kern = pl.kernel(
        functools.partial(_gather_body, n_chunks_total=n_rows // CHUNK),
        out_type=jax.ShapeDtypeStruct((n_rows // IDX_W, IDX_W, EMBED_DIM),
                                      jnp.float32),
        mesh=mesh,
        scratch_types=[
            pltpu.VMEM((CHUNK,), jnp.int32),
            pltpu.VMEM((CHUNK,), jnp.int32),
            pltpu.VMEM((K, IDX_W, EMBED_DIM), jnp.float32),
            pltpu.SemaphoreType.DMA,
        ],
        compiler_params=pltpu.CompilerParams(use_tc_tiling_on_sc=False),
    )
    return kern(table_rows, idx_flat)


def _mm_body(emb_ref, w_ref, b_ref, *rest):
    out_ref = rest[-1]
    halves = []
    for h in range(2):
        strips = []
        for u in range(PACK):
            p_u = emb_ref[h * (CHUNK // PACK):(h + 1) * (CHUNK // PACK),
                          u * EMBED_DIM:(u + 1) * EMBED_DIM]
            strips.append(
                lax.dot_general(w_ref[...], p_u, (((1,), (1,)), ((), ())),
                                preferred_element_type=jnp.float32))
        halves.extend(strips)
    acc = jnp.concatenate(halves, axis=1)
    out_ref[...] = jnp.maximum(acc + b_ref[:, :1], 0.0)[None]


def _tc_linear_relu_t(emb_packed, Wt, bcol, seq, batch, s_off, total_seq,
                      prev=None):
    grid = (seq,)
    in_specs = [
        pl.BlockSpec((batch // PACK, LANES), lambda s: (s, 0)),
        pl.BlockSpec((D_MODEL, EMBED_DIM), lambda s: (0, 0)),
        pl.BlockSpec((D_MODEL, LANES), lambda s: (0, 0)),
    ]
    args = [emb_packed, Wt, bcol]
    aliases = {}
    if prev is not None:
        in_specs.append(pl.BlockSpec(memory_space=pl.ANY))
        args.append(prev)
        aliases = {3: 0}
    return pl.pallas_call(
        _mm_body,
        grid=grid,
        in_specs=in_specs,
        out_specs=pl.BlockSpec((1, D_MODEL, batch),
                               lambda s: (s + s_off, 0, 0)),
        out_shape=jax.ShapeDtypeStruct((total_seq, D_MODEL, batch),
                                       jnp.float32),
        input_output_aliases=aliases,
    )(*args)


def kernel(x, table, W, b):
    batch, seq = x.shape
    n_rows = batch * seq
    vocab = table.shape[0]
    v_pad = ((vocab + TBLK - 1) // TBLK) * TBLK

    table_lin = _tc_table_linear(table.T, v_pad)
    table_rows = table_lin.reshape(v_pad, EMBED_DIM)

    v = x.T.astype(jnp.int32).reshape(n_rows)
    addr = (v & -VBLK) | ((v & (VBLK // PACK - 1)) << 2) | ((v & (VBLK - 1))
                                                           >> 10)
    n_split = 5
    seq_q = seq // n_split
    rows_q = n_rows // n_split
    Wt = W.T
    bcol = jnp.tile(b.reshape(D_MODEL, 1), (1, LANES))
    out_t = None
    for i in range(n_split):
        addr_i = lax.slice(addr, (i * rows_q,), ((i + 1) * rows_q,))
        emb3 = _sc_gather(table_rows, addr_i, rows_q)
        emb_packed = emb3.reshape(rows_q * EMBED_DIM // LANES, LANES)
        out_t = _tc_linear_relu_t(emb_packed, Wt, bcol, seq_q, batch,
                                  i * seq_q, seq, prev=out_t)
    return lax.transpose(out_t, (2, 0, 1))
